# baseline (speedup 1.0000x reference)
import functools

import jax
import jax.numpy as jnp
from jax import lax
from jax.experimental import pallas as pl
from jax.experimental.pallas import tpu as pltpu
from jax.experimental.pallas import tpu_sc as plsc

F = 26
V = 40000
B = 16384
L = 16
H = B // 2
R = H // 16


def _fm_body(x_hbm, w_hbm, out_hbm,
             xflat_v, xcol_v, tab_v, fidx_v, local_v, row_v, acc_v,
             xt_sh, sums_sh):
    cid = lax.axis_index("c")
    sid = lax.axis_index("s")
    row0 = cid * H + sid * R

    pltpu.sync_copy(x_hbm.at[pl.ds(row0 * F, R * F)], xflat_v)
    lanes26 = lax.iota(jnp.int32, L) * F

    def transpose_chunk(c, carry):
        base = c * (L * F)
        for f in range(F):
            flat = base + f + lanes26
            xcol_v[f, pl.ds(c * L, L)] = plsc.load_gather(xflat_v, [flat])
        return carry

    lax.fori_loop(0, R // L, transpose_chunk, 0)
    for f in range(F):
        pltpu.sync_copy(xcol_v.at[f], xt_sh.at[f, pl.ds(sid * R, R)])

    plsc.subcore_barrier()

    for k in range(2):
        gf = sid + 16 * k

        @pl.when(gf < F)
        def _():
            pltpu.sync_copy(w_hbm.at[pl.ds(gf * V, V)], tab_v)
            pltpu.sync_copy(xt_sh.at[gf], fidx_v)

            def gather_chunk(c, carry):
                g = plsc.load_gather(tab_v, [fidx_v[pl.ds(c * L, L)]])
                if k == 0:
                    local_v[pl.ds(c * L, L)] = g
                else:
                    local_v[pl.ds(c * L, L)] = local_v[pl.ds(c * L, L)] + g
                return carry

            lax.fori_loop(0, H // L, gather_chunk, 0)

    pltpu.sync_copy(local_v, sums_sh.at[sid])
    plsc.subcore_barrier()

    for j in range(16):
        pltpu.sync_copy(sums_sh.at[j, pl.ds(sid * R, R)], row_v)

        def add_chunk(c, carry):
            if j == 0:
                acc_v[pl.ds(c * L, L)] = row_v[pl.ds(c * L, L)]
            else:
                acc_v[pl.ds(c * L, L)] = acc_v[pl.ds(c * L, L)] + row_v[pl.ds(c * L, L)]
            return carry

        lax.fori_loop(0, R // L, add_chunk, 0)

    pltpu.sync_copy(acc_v, out_hbm.at[pl.ds(row0, R)])


_fm_sc = functools.partial(
    pl.kernel,
    out_type=jax.ShapeDtypeStruct((B,), jnp.float32),
    mesh=plsc.VectorSubcoreMesh(core_axis_name="c", subcore_axis_name="s"),
    compiler_params=pltpu.CompilerParams(needs_layout_passes=False),
    scratch_types=[
        pltpu.VMEM((R * F,), jnp.int32),
        pltpu.VMEM((F, R), jnp.int32),
        pltpu.VMEM((V,), jnp.float32),
        pltpu.VMEM((H,), jnp.int32),
        pltpu.VMEM((H,), jnp.float32),
        pltpu.VMEM((R,), jnp.float32),
        pltpu.VMEM((R,), jnp.float32),
        pltpu.VMEM_SHARED((F, H), jnp.int32),
        pltpu.VMEM_SHARED((16, H), jnp.float32),
    ],
)(_fm_body)


@jax.jit
def kernel(x, W):
    out = _fm_sc(x.reshape(-1), W.reshape(-1))
    return out.reshape(B, 1)


# speedup vs baseline: 2.6638x; 2.6638x over previous
import functools

import jax
import jax.numpy as jnp
from jax import lax
from jax.experimental import pallas as pl
from jax.experimental.pallas import tpu as pltpu
from jax.experimental.pallas import tpu_sc as plsc

F = 26
V = 40000
B = 16384
L = 16
H = B // 2
R = H // 16
TAB = 40064


def _fm_body(xt_hbm, wt_hbm, out_hbm,
             tab_v, fidx_v, local_v, row_v, acc_v, sums_sh):
    cid = lax.axis_index("c")
    sid = lax.axis_index("s")

    for k in range(2):
        gf = sid + 16 * k

        @pl.when(gf < F)
        def _():
            d = (gf % 2) * 64
            r0 = pl.multiple_of(gf * V - d, 128)
            pltpu.sync_copy(wt_hbm.at[0, pl.ds(r0, TAB)], tab_v)
            pltpu.sync_copy(xt_hbm.at[gf, pl.ds(cid * H, H)], fidx_v)

            def gather_chunk(c, carry):
                idx = fidx_v[pl.ds(c * L, L)] + d
                g = plsc.load_gather(tab_v, [idx])
                if k == 0:
                    local_v[pl.ds(c * L, L)] = g
                else:
                    local_v[pl.ds(c * L, L)] = local_v[pl.ds(c * L, L)] + g
                return carry

            lax.fori_loop(0, H // L, gather_chunk, 0)

    pltpu.sync_copy(local_v, sums_sh.at[sid])
    plsc.subcore_barrier()

    for j in range(16):
        pltpu.sync_copy(sums_sh.at[j, pl.ds(sid * R, R)], row_v)

        def add_chunk(c, carry):
            if j == 0:
                acc_v[pl.ds(c * L, L)] = row_v[pl.ds(c * L, L)]
            else:
                acc_v[pl.ds(c * L, L)] = acc_v[pl.ds(c * L, L)] + row_v[pl.ds(c * L, L)]
            return carry

        lax.fori_loop(0, R // L, add_chunk, 0)

    pltpu.sync_copy(acc_v, out_hbm.at[pl.ds(cid * H + sid * R, R)])


_fm_sc = functools.partial(
    pl.kernel,
    out_type=jax.ShapeDtypeStruct((B,), jnp.float32),
    mesh=plsc.VectorSubcoreMesh(core_axis_name="c", subcore_axis_name="s"),
    compiler_params=pltpu.CompilerParams(needs_layout_passes=False),
    scratch_types=[
        pltpu.VMEM((TAB,), jnp.float32),
        pltpu.VMEM((H,), jnp.int32),
        pltpu.VMEM((H,), jnp.float32),
        pltpu.VMEM((R,), jnp.float32),
        pltpu.VMEM((R,), jnp.float32),
        pltpu.VMEM_SHARED((16, H), jnp.float32),
    ],
)(_fm_body)


@jax.jit
def kernel(x, W):
    out = _fm_sc(x.T, W.T)
    return out.reshape(B, 1)


# speedup vs baseline: 2.9478x; 1.1066x over previous
import functools

import jax
import jax.numpy as jnp
from jax import lax
from jax.experimental import pallas as pl
from jax.experimental.pallas import tpu as pltpu
from jax.experimental.pallas import tpu_sc as plsc

F = 26
V = 40000
B = 16384
L = 16
H = B // 2
R = H // 16
TAB = 40064
UNROLL = 4


def _fm_body(xt_hbm, wt_hbm, out_hbm,
             taba_v, tabb_v, fidxa_v, fidxb_v, local_v, red_v, acc_v, sums_sh,
             sem_ta, sem_tb, sem_ia, sem_ib, sem_r):
    cid = lax.axis_index("c")
    sid = lax.axis_index("s")
    tabs = (taba_v, tabb_v)
    fidxs = (fidxa_v, fidxb_v)
    sems = ((sem_ta, sem_ia), (sem_tb, sem_ib))

    copies = []
    for k in range(2):
        gf = sid + 16 * k

        @pl.when(gf < F)
        def _():
            d = (gf % 2) * 64
            r0 = pl.multiple_of(gf * V - d, 128)
            pltpu.async_copy(wt_hbm.at[0, pl.ds(r0, TAB)], tabs[k], sems[k][0])
            pltpu.async_copy(xt_hbm.at[gf, pl.ds(cid * H, H)], fidxs[k], sems[k][1])

    for k in range(2):
        gf = sid + 16 * k

        @pl.when(gf < F)
        def _():
            d = (gf % 2) * 64
            pltpu.make_async_copy(wt_hbm.at[0, pl.ds(0, TAB)], tabs[k], sems[k][0]).wait()
            pltpu.make_async_copy(xt_hbm.at[0, pl.ds(0, H)], fidxs[k], sems[k][1]).wait()
            tab_v = tabs[k]
            fidx_v = fidxs[k]

            def gather_chunk(c, carry):
                for u in range(UNROLL):
                    o = (c * UNROLL + u) * L
                    idx = fidx_v[pl.ds(o, L)] + d
                    g = plsc.load_gather(tab_v, [idx])
                    if k == 0:
                        local_v[pl.ds(o, L)] = g
                    else:
                        local_v[pl.ds(o, L)] = local_v[pl.ds(o, L)] + g
                return carry

            lax.fori_loop(0, H // (L * UNROLL), gather_chunk, 0)

    pltpu.sync_copy(local_v, sums_sh.at[sid])
    plsc.subcore_barrier()

    for j in range(16):
        pltpu.async_copy(sums_sh.at[j, pl.ds(sid * R, R)], red_v.at[j], sem_r).wait()

    def add_chunk(c, carry):
        o = c * L
        acc = red_v[0, pl.ds(o, L)]
        for j in range(1, 16):
            acc = acc + red_v[j, pl.ds(o, L)]
        acc_v[pl.ds(o, L)] = acc
        return carry

    lax.fori_loop(0, R // L, add_chunk, 0)

    pltpu.sync_copy(acc_v, out_hbm.at[pl.ds(cid * H + sid * R, R)])


_fm_sc = functools.partial(
    pl.kernel,
    out_type=jax.ShapeDtypeStruct((B,), jnp.float32),
    mesh=plsc.VectorSubcoreMesh(core_axis_name="c", subcore_axis_name="s"),
    compiler_params=pltpu.CompilerParams(needs_layout_passes=False),
    scratch_types=[
        pltpu.VMEM((TAB,), jnp.float32),
        pltpu.VMEM((TAB,), jnp.float32),
        pltpu.VMEM((H,), jnp.int32),
        pltpu.VMEM((H,), jnp.int32),
        pltpu.VMEM((H,), jnp.float32),
        pltpu.VMEM((16, R), jnp.float32),
        pltpu.VMEM((R,), jnp.float32),
        pltpu.VMEM_SHARED((16, H), jnp.float32),
        pltpu.SemaphoreType.DMA,
        pltpu.SemaphoreType.DMA,
        pltpu.SemaphoreType.DMA,
        pltpu.SemaphoreType.DMA,
        pltpu.SemaphoreType.DMA,
    ],
)(_fm_body)


@jax.jit
def kernel(x, W):
    out = _fm_sc(x.T, W.T)
    return out.reshape(B, 1)


# speedup vs baseline: 3.4973x; 1.1864x over previous
import functools

import jax
import jax.numpy as jnp
from jax import lax
from jax.experimental import pallas as pl
from jax.experimental.pallas import tpu as pltpu
from jax.experimental.pallas import tpu_sc as plsc

F = 26
V = 40000
B = 16384
L = 16
H = B // 2
R = H // 16
TAB = 40064
UNROLL = 4


def _fm_body(xt_hbm, wt_hbm, out_hbm,
             taba_v, tabb_v, fidxa_v, fidxb_v, local_v, red_v, acc_v, sums_sh,
             sem_ta, sem_tb, sem_ia, sem_ib, sem_r):
    cid = lax.axis_index("c")
    sid = lax.axis_index("s")
    tabs = (taba_v, tabb_v)
    fidxs = (fidxa_v, fidxb_v)
    sems = ((sem_ta, sem_ia), (sem_tb, sem_ib))

    copies = []
    for k in range(2):
        gf = sid + 16 * k

        @pl.when(gf < F)
        def _():
            d = (gf % 2) * 64
            r0 = pl.multiple_of(gf * V - d, 128)
            pltpu.async_copy(wt_hbm.at[0, pl.ds(r0, TAB)], tabs[k], sems[k][0])
            pltpu.async_copy(xt_hbm.at[gf, pl.ds(cid * H, H)], fidxs[k], sems[k][1])

    for k in range(2):
        gf = sid + 16 * k

        @pl.when(gf < F)
        def _():
            d = (gf % 2) * 64
            pltpu.make_async_copy(wt_hbm.at[0, pl.ds(0, TAB)], tabs[k], sems[k][0]).wait()
            pltpu.make_async_copy(xt_hbm.at[0, pl.ds(0, H)], fidxs[k], sems[k][1]).wait()
            tab_v = tabs[k]
            fidx_v = fidxs[k]

            @plsc.parallel_loop(0, H // L, unroll=UNROLL)
            def gather_chunk(c):
                o = c * L
                idx = fidx_v[pl.ds(o, L)] + d
                g = plsc.load_gather(tab_v, [idx])
                if k == 0:
                    local_v[pl.ds(o, L)] = g
                else:
                    local_v[pl.ds(o, L)] = local_v[pl.ds(o, L)] + g

    pltpu.sync_copy(local_v, sums_sh.at[sid])
    plsc.subcore_barrier()

    row_copies = [
        pltpu.async_copy(sums_sh.at[j, pl.ds(sid * R, R)], red_v.at[j], sem_r)
        for j in range(16)
    ]
    for c in row_copies:
        c.wait()

    @plsc.parallel_loop(0, R // L, unroll=2)
    def add_chunk(c):
        o = c * L
        acc = red_v[0, pl.ds(o, L)]
        for j in range(1, 16):
            acc = acc + red_v[j, pl.ds(o, L)]
        acc_v[pl.ds(o, L)] = acc

    pltpu.sync_copy(acc_v, out_hbm.at[pl.ds(cid * H + sid * R, R)])


_fm_sc = functools.partial(
    pl.kernel,
    out_type=jax.ShapeDtypeStruct((B,), jnp.float32),
    mesh=plsc.VectorSubcoreMesh(core_axis_name="c", subcore_axis_name="s"),
    compiler_params=pltpu.CompilerParams(needs_layout_passes=False),
    scratch_types=[
        pltpu.VMEM((TAB,), jnp.float32),
        pltpu.VMEM((TAB,), jnp.float32),
        pltpu.VMEM((H,), jnp.int32),
        pltpu.VMEM((H,), jnp.int32),
        pltpu.VMEM((H,), jnp.float32),
        pltpu.VMEM((16, R), jnp.float32),
        pltpu.VMEM((R,), jnp.float32),
        pltpu.VMEM_SHARED((16, H), jnp.float32),
        pltpu.SemaphoreType.DMA,
        pltpu.SemaphoreType.DMA,
        pltpu.SemaphoreType.DMA,
        pltpu.SemaphoreType.DMA,
        pltpu.SemaphoreType.DMA,
    ],
)(_fm_body)


@jax.jit
def kernel(x, W):
    out = _fm_sc(x.T, W.T)
    return out.reshape(B, 1)
